# Initial kernel scaffold; baseline (speedup 1.0000x reference)
import jax
import jax.numpy as jnp
from jax.experimental import pallas as pl


def kernel(bfs_x, bf_x, bfs_pre_h, bf_pre_h, edge_index, edge_attr, params):
    raise NotImplementedError("write your pallas kernel here")


# speedup vs baseline: 2.2380x; 2.2380x over previous
import functools

import jax
import jax.numpy as jnp
from jax import lax
from jax.experimental import pallas as pl
from jax.experimental.pallas import tpu as pltpu
from jax.experimental.pallas import tpu_sc as plsc

N = 10000
E = 320000
H = 128
BN = 1024
GRID = (N + BN - 1) // BN
CHUNK = 4000
NCHUNK = E // CHUNK
NGROUP = CHUNK // 16
NEG = float('-inf')
_P = jax.lax.Precision.HIGHEST


def _tc_pre_body(xb_ref, xf_ref, hb_ref, hf_ref,
                 ew0b_ref, ewhb_ref, ebb_ref, ew0f_ref, ewhf_ref, ebf_ref,
                 mw1_ref, mw2_ref, mbt_ref,
                 zb_ref, zf_ref, abt_b_ref, abt_f_ref, btall_ref):
    xb = xb_ref[...]
    xf = xf_ref[...]
    hb = hb_ref[...]
    hf = hf_ref[...]
    zb = jnp.maximum(xb * ew0b_ref[...] + jnp.dot(hb, ewhb_ref[...], precision=_P)
                     + ebb_ref[...], 0.0)
    zf = jnp.maximum(xf * ew0f_ref[...] + jnp.dot(hf, ewhf_ref[...], precision=_P)
                     + ebf_ref[...], 0.0)
    zb_ref[...] = zb
    zf_ref[...] = zf
    mw1 = mw1_ref[...]
    mw2 = mw2_ref[...]
    mbt = mbt_ref[...]
    abt_b_ref[...] = lax.dot_general(mw1, zb, (((0,), (1,)), ((), ())),
                                     precision=_P) + mbt
    abt_f_ref[...] = lax.dot_general(mw1, zf, (((0,), (1,)), ((), ())),
                                     precision=_P) + mbt
    bt_b = lax.dot_general(mw2, zb, (((0,), (1,)), ((), ())), precision=_P)
    bt_f = lax.dot_general(mw2, zf, (((0,), (1,)), ((), ())), precision=_P)
    btall_ref[...] = jnp.concatenate([bt_b, bt_f], axis=0)


def _tc_pre(bfs_x, bf_x, bfs_pre_h, bf_pre_h, ew0b, ewhb, ebb, ew0f, ewhf, ebf,
            mw1, mw2, mbt):
    row_spec1 = pl.BlockSpec((BN, 1), lambda i: (i, 0))
    row_specH = pl.BlockSpec((BN, H), lambda i: (i, 0))
    full = lambda shape: pl.BlockSpec(shape, lambda i: tuple(0 for _ in shape))
    colT_spec = pl.BlockSpec((H, BN), lambda i: (0, i))
    colT2_spec = pl.BlockSpec((2 * H, BN), lambda i: (0, i))
    return pl.pallas_call(
        _tc_pre_body,
        grid=(GRID,),
        in_specs=[row_spec1, row_spec1, row_specH, row_specH,
                  full((1, H)), full((H, H)), full((1, H)),
                  full((1, H)), full((H, H)), full((1, H)),
                  full((H, H)), full((H, H)), full((H, 1))],
        out_specs=[row_specH, row_specH, colT_spec, colT_spec, colT2_spec],
        out_shape=[jax.ShapeDtypeStruct((N, H), jnp.float32),
                   jax.ShapeDtypeStruct((N, H), jnp.float32),
                   jax.ShapeDtypeStruct((H, N), jnp.float32),
                   jax.ShapeDtypeStruct((H, N), jnp.float32),
                   jax.ShapeDtypeStruct((2 * H, N), jnp.float32)],
    )(bfs_x, bf_x, bfs_pre_h, bf_pre_h, ew0b, ewhb, ebb, ew0f, ewhf, ebf,
      mw1, mw2, mbt)


def _sc_body(bt_hbm, dst_hbm, src_hbm, attr_hbm, w_hbm, out_hbm,
             bv, acc, tmp, dch, sch, ach, wv):
    wid = lax.axis_index("s") * 2 + lax.axis_index("c")
    pltpu.sync_copy(w_hbm, wv)
    lane = lax.iota(jnp.int32, 16)

    for half in range(2):
        r0 = wid * 8 + half * 4
        for j in range(4):
            pltpu.sync_copy(bt_hbm.at[pl.ds((r0 + j) * N, N)],
                            bv.at[pl.ds(j * N, N)])

        def _init(k, _):
            acc[pl.ds(k * 16, 16)] = jnp.full((16,), NEG, jnp.float32)
            return 0
        lax.fori_loop(0, 4 * N // 16, _init, 0)

        wspl = [plsc.load_gather(wv, [jnp.broadcast_to(r0 + j, (16,)).astype(jnp.int32)])
                for j in range(4)]

        def _chunk(ci, _):
            off = ci * CHUNK
            pltpu.sync_copy(dst_hbm.at[pl.ds(off, CHUNK)], dch)
            pltpu.sync_copy(src_hbm.at[pl.ds(off, CHUNK)], sch)
            pltpu.sync_copy(attr_hbm.at[pl.ds(off, CHUNK)], ach)

            def _group(g, _):
                base = g * 16
                dvec = dch[pl.ds(base, 16)]
                svec = sch[pl.ds(base, 16)]
                avec = ach[pl.ds(base, 16)]
                plsc.store_scatter(tmp, [dvec], lane)
                win = plsc.load_gather(tmp, [dvec]) == lane
                vals = []
                dvecs = []
                for j in range(4):
                    svecj = svec + jnp.int32(j * N)
                    dvecj = dvec + jnp.int32(j * N)
                    dvecs.append(dvecj)
                    bvj = plsc.load_gather(bv, [svecj])
                    valj = bvj + avec * wspl[j]
                    vals.append(valj)
                    accv = plsc.load_gather(acc, [dvecj])
                    plsc.store_scatter(acc, [dvecj],
                                       jnp.maximum(accv, valj), mask=win)
                lose = jnp.logical_not(win)

                @pl.when(jnp.any(lose))
                def _():
                    def _cond(m):
                        return jnp.any(m)

                    def _body(m):
                        plsc.store_scatter(tmp, [dvec], lane, mask=m)
                        w2 = jnp.logical_and(
                            m, plsc.load_gather(tmp, [dvec]) == lane)
                        for j in range(4):
                            accv2 = plsc.load_gather(acc, [dvecs[j]])
                            plsc.store_scatter(acc, [dvecs[j]],
                                               jnp.maximum(accv2, vals[j]),
                                               mask=w2)
                        return jnp.logical_and(m, jnp.logical_not(w2))
                    lax.while_loop(_cond, _body, lose)
                return 0
            lax.fori_loop(0, NGROUP, _group, 0)
            return 0
        lax.fori_loop(0, NCHUNK, _chunk, 0)
        for j in range(4):
            pltpu.sync_copy(acc.at[pl.ds(j * N, N)],
                            out_hbm.at[pl.ds((r0 + j) * N, N)])


def _sc_segmax(btall, dst, src, attr, wall):
    mesh = plsc.VectorSubcoreMesh(core_axis_name="c", subcore_axis_name="s")
    return pl.kernel(
        _sc_body,
        out_type=jax.ShapeDtypeStruct((2 * H * N,), jnp.float32),
        mesh=mesh,
        compiler_params=pltpu.CompilerParams(needs_layout_passes=False),
        scratch_types=[
            pltpu.VMEM((4 * N,), jnp.float32),
            pltpu.VMEM((4 * N,), jnp.float32),
            pltpu.VMEM((N,), jnp.int32),
            pltpu.VMEM((CHUNK,), jnp.int32),
            pltpu.VMEM((CHUNK,), jnp.int32),
            pltpu.VMEM((CHUNK,), jnp.float32),
            pltpu.VMEM((2 * H,), jnp.float32),
        ],
    )(btall, dst, src, attr, wall)


def _tc_post_body(zb_ref, zf_ref, abt_b_ref, abt_f_ref, seg_ref,
                  u1_ref, u2_ref, ub_ref,
                  dzb_ref, dhb_ref, dbb_ref, dzf_ref, dhf_ref, dbf_ref,
                  twb_ref, tbb_ref, twf_ref, tbf_ref,
                  h_ref, y_ref, dist_ref, taub_ref, tauf_ref, hsb_ref, hsf_ref):
    i = pl.program_id(0)
    zb = zb_ref[...]
    zf = zf_ref[...]
    seg = seg_ref[...]
    aggrT_b = jnp.maximum(abt_b_ref[...] + seg[:H, :], 0.0)
    aggrT_f = jnp.maximum(abt_f_ref[...] + seg[H:, :], 0.0)
    u1 = u1_ref[...]
    u2 = u2_ref[...]
    ub = ub_ref[...]
    hb = jnp.maximum(jnp.dot(zb, u1, precision=_P)
                     + lax.dot_general(aggrT_b, u2, (((0,), (0,)), ((), ())),
                                       precision=_P) + ub, 0.0)
    hf = jnp.maximum(jnp.dot(zf, u1, precision=_P)
                     + lax.dot_general(aggrT_f, u2, (((0,), (0,)), ((), ())),
                                       precision=_P) + ub, 0.0)
    h_ref[...] = hb
    y_ref[...] = (jnp.dot(zb, dzb_ref[...], precision=_P)
                  + jnp.dot(hb, dhb_ref[...], precision=_P) + dbb_ref[...])
    dist_ref[...] = (jnp.dot(zf, dzf_ref[...], precision=_P)
                     + jnp.dot(hf, dhf_ref[...], precision=_P) + dbf_ref[...])

    valid = jnp.minimum(N - i * BN, BN)
    rmask = lax.broadcasted_iota(jnp.int32, (BN, 1), 0) < valid
    sb = jnp.sum(jnp.where(rmask, hb, 0.0), axis=0, keepdims=True)
    sf = jnp.sum(jnp.where(rmask, hf, 0.0), axis=0, keepdims=True)
    pad = jnp.zeros((7, H), jnp.float32)
    sb8 = jnp.concatenate([sb, pad], axis=0)
    sf8 = jnp.concatenate([sf, pad], axis=0)

    @pl.when(i == 0)
    def _():
        hsb_ref[...] = jnp.zeros((8, H), jnp.float32)
        hsf_ref[...] = jnp.zeros((8, H), jnp.float32)

    hsb_ref[...] += sb8
    hsf_ref[...] += sf8

    @pl.when(i == GRID - 1)
    def _():
        hmb = hsb_ref[0:1, :] * jnp.float32(1.0 / N)
        hmf = hsf_ref[0:1, :] * jnp.float32(1.0 / N)
        taub_ref[...] = jax.nn.sigmoid(
            jnp.dot(hmb, twb_ref[...], precision=_P) + tbb_ref[...])
        tauf_ref[...] = jax.nn.sigmoid(
            jnp.dot(hmf, twf_ref[...], precision=_P) + tbf_ref[...])


def _tc_post(zb, zf, abt_b, abt_f, seg, u1, u2, ub,
             dzb, dhb, dbb, dzf, dhf, dbf, twb, tbb, twf, tbf):
    row_specH = pl.BlockSpec((BN, H), lambda i: (i, 0))
    row_spec1 = pl.BlockSpec((BN, 1), lambda i: (i, 0))
    colT_spec = pl.BlockSpec((H, BN), lambda i: (0, i))
    colT2_spec = pl.BlockSpec((2 * H, BN), lambda i: (0, i))
    full = lambda shape: pl.BlockSpec(shape, lambda i: tuple(0 for _ in shape))
    return pl.pallas_call(
        _tc_post_body,
        grid=(GRID,),
        in_specs=[row_specH, row_specH, colT_spec, colT_spec, colT2_spec,
                  full((H, H)), full((H, H)), full((1, H)),
                  full((H, 1)), full((H, 1)), full((1, 1)),
                  full((H, 1)), full((H, 1)), full((1, 1)),
                  full((H, 1)), full((1, 1)), full((H, 1)), full((1, 1))],
        out_specs=[row_specH, row_spec1, row_spec1,
                   full((1, 1)), full((1, 1)), full((8, H)), full((8, H))],
        out_shape=[jax.ShapeDtypeStruct((N, H), jnp.float32),
                   jax.ShapeDtypeStruct((N, 1), jnp.float32),
                   jax.ShapeDtypeStruct((N, 1), jnp.float32),
                   jax.ShapeDtypeStruct((1, 1), jnp.float32),
                   jax.ShapeDtypeStruct((1, 1), jnp.float32),
                   jax.ShapeDtypeStruct((8, H), jnp.float32),
                   jax.ShapeDtypeStruct((8, H), jnp.float32)],
    )(zb, zf, abt_b, abt_f, seg, u1, u2, ub,
      dzb, dhb, dbb, dzf, dhf, dbf, twb, tbb, twf, tbf)


def kernel(bfs_x, bf_x, bfs_pre_h, bf_pre_h, edge_index, edge_attr, params):
    src = edge_index[0].astype(jnp.int32)
    dst = edge_index[1].astype(jnp.int32)
    attr = edge_attr.reshape(E).astype(jnp.float32)

    p = params
    ew0b = p['enc_bfs_W'][0:1]
    ewhb = p['enc_bfs_W'][1:]
    ebb = p['enc_bfs_b'].reshape(1, H)
    ew0f = p['enc_bf_W'][0:1]
    ewhf = p['enc_bf_W'][1:]
    ebf = p['enc_bf_b'].reshape(1, H)
    mw1 = p['M_W'][0:H]
    mw2 = p['M_W'][H:2 * H]
    wall = jnp.concatenate([p['M_W'][2 * H], p['M_W'][2 * H]], axis=0)
    mbt = p['M_b'].reshape(H, 1)
    u1 = p['U_W'][0:H]
    u2 = p['U_W'][H:]
    ub = p['U_b'].reshape(1, H)
    dzb = p['dec_bfs_W'][0:H]
    dhb = p['dec_bfs_W'][H:]
    dbb = p['dec_bfs_b'].reshape(1, 1)
    dzf = p['dec_bf_W'][0:H]
    dhf = p['dec_bf_W'][H:]
    dbf = p['dec_bf_b'].reshape(1, 1)
    twb = p['term_bfs_W'][0:H] + p['term_bfs_W'][H:]
    tbb = p['term_bfs_b'].reshape(1, 1)
    twf = p['term_bf_W'][0:H] + p['term_bf_W'][H:]
    tbf = p['term_bf_b'].reshape(1, 1)

    zb, zf, abt_b, abt_f, btall = _tc_pre(
        bfs_x, bf_x, bfs_pre_h, bf_pre_h,
        ew0b, ewhb, ebb, ew0f, ewhf, ebf, mw1, mw2, mbt)

    seg = _sc_segmax(btall.reshape(2 * H * N), dst, src, attr, wall)
    seg = seg.reshape(2 * H, N)

    bfs_h, y, dist, taub, tauf, _, _ = _tc_post(
        zb, zf, abt_b, abt_f, seg, u1, u2, ub,
        dzb, dhb, dbb, dzf, dhf, dbf, twb, tbb, twf, tbf)

    return (bfs_h, bfs_h, y, dist, taub, tauf)


# speedup vs baseline: 2.7603x; 1.2334x over previous
import functools

import jax
import jax.numpy as jnp
from jax import lax
from jax.experimental import pallas as pl
from jax.experimental.pallas import tpu as pltpu
from jax.experimental.pallas import tpu_sc as plsc

N = 10000
E = 320000
H = 128
BN = 1024
GRID = (N + BN - 1) // BN
CHUNK = 4000
NCHUNK = E // CHUNK
NGROUP = CHUNK // 16
NEG = float('-inf')
_P = jax.lax.Precision.HIGHEST


def _tc_pre_body(xb_ref, xf_ref, hb_ref, hf_ref,
                 ew0b_ref, ewhb_ref, ebb_ref, ew0f_ref, ewhf_ref, ebf_ref,
                 mw1_ref, mw2_ref, mbt_ref,
                 zb_ref, zf_ref, abt_b_ref, abt_f_ref, btall_ref):
    xb = xb_ref[...]
    xf = xf_ref[...]
    hb = hb_ref[...]
    hf = hf_ref[...]
    zb = jnp.maximum(xb * ew0b_ref[...] + jnp.dot(hb, ewhb_ref[...], precision=_P)
                     + ebb_ref[...], 0.0)
    zf = jnp.maximum(xf * ew0f_ref[...] + jnp.dot(hf, ewhf_ref[...], precision=_P)
                     + ebf_ref[...], 0.0)
    zb_ref[...] = zb
    zf_ref[...] = zf
    mw1 = mw1_ref[...]
    mw2 = mw2_ref[...]
    mbt = mbt_ref[...]
    abt_b_ref[...] = lax.dot_general(mw1, zb, (((0,), (1,)), ((), ())),
                                     precision=_P) + mbt
    abt_f_ref[...] = lax.dot_general(mw1, zf, (((0,), (1,)), ((), ())),
                                     precision=_P) + mbt
    bt_b = lax.dot_general(mw2, zb, (((0,), (1,)), ((), ())), precision=_P)
    bt_f = lax.dot_general(mw2, zf, (((0,), (1,)), ((), ())), precision=_P)
    btall_ref[...] = jnp.concatenate([bt_b, bt_f], axis=0)


def _tc_pre(bfs_x, bf_x, bfs_pre_h, bf_pre_h, ew0b, ewhb, ebb, ew0f, ewhf, ebf,
            mw1, mw2, mbt):
    row_spec1 = pl.BlockSpec((BN, 1), lambda i: (i, 0))
    row_specH = pl.BlockSpec((BN, H), lambda i: (i, 0))
    full = lambda shape: pl.BlockSpec(shape, lambda i: tuple(0 for _ in shape))
    colT_spec = pl.BlockSpec((H, BN), lambda i: (0, i))
    colT2_spec = pl.BlockSpec((2 * H, BN), lambda i: (0, i))
    return pl.pallas_call(
        _tc_pre_body,
        grid=(GRID,),
        in_specs=[row_spec1, row_spec1, row_specH, row_specH,
                  full((1, H)), full((H, H)), full((1, H)),
                  full((1, H)), full((H, H)), full((1, H)),
                  full((H, H)), full((H, H)), full((H, 1))],
        out_specs=[row_specH, row_specH, colT_spec, colT_spec, colT2_spec],
        out_shape=[jax.ShapeDtypeStruct((N, H), jnp.float32),
                   jax.ShapeDtypeStruct((N, H), jnp.float32),
                   jax.ShapeDtypeStruct((H, N), jnp.float32),
                   jax.ShapeDtypeStruct((H, N), jnp.float32),
                   jax.ShapeDtypeStruct((2 * H, N), jnp.float32)],
    )(bfs_x, bf_x, bfs_pre_h, bf_pre_h, ew0b, ewhb, ebb, ew0f, ewhf, ebf,
      mw1, mw2, mbt)


def _sc_body(bt_hbm, edges_hbm, w_hbm, out_hbm,
             bv0, bv1, bv2, bv3, acc0, acc1, acc2, acc3,
             tmp, eb0, eb1, wv, sem0, sem1):
    wid = lax.axis_index("s") * 2 + lax.axis_index("c")
    pltpu.sync_copy(w_hbm, wv)
    lane = lax.iota(jnp.int32, 16)
    bvs = [bv0, bv1, bv2, bv3]
    accs = [acc0, acc1, acc2, acc3]
    ebufs = [eb0, eb1]
    sems = [sem0, sem1]
    W3 = 3 * CHUNK

    def _start(ci, b):
        pltpu.async_copy(edges_hbm.at[pl.ds(ci * W3, W3)], ebufs[b], sems[b])

    def _wait(b):
        pltpu.make_async_copy(edges_hbm.at[pl.ds(0, W3)], ebufs[b],
                              sems[b]).wait()

    for half in range(2):
        r0 = wid * 8 + half * 4
        for j in range(4):
            pltpu.sync_copy(bt_hbm.at[pl.ds((r0 + j) * N, N)], bvs[j])

        def _init(k, _):
            for j in range(4):
                accs[j][pl.ds(k * 16, 16)] = jnp.full((16,), NEG, jnp.float32)
            return 0
        lax.fori_loop(0, N // 16, _init, 0)

        wspl = [plsc.load_gather(wv, [jnp.broadcast_to(r0 + j, (16,)).astype(jnp.int32)])
                for j in range(4)]

        def _process(b):
            eb = ebufs[b]

            def _group(g, _):
                base = g * 16
                svec = eb[pl.ds(base, 16)]
                dvec = eb[pl.ds(CHUNK + base, 16)]
                avec = plsc.bitcast(eb[pl.ds(2 * CHUNK + base, 16)],
                                    jnp.float32)
                plsc.store_scatter(tmp, [dvec], lane)
                win = plsc.load_gather(tmp, [dvec]) == lane
                vals = []
                for j in range(4):
                    bvj = plsc.load_gather(bvs[j], [svec])
                    valj = bvj + avec * wspl[j]
                    vals.append(valj)
                    accv = plsc.load_gather(accs[j], [dvec])
                    plsc.store_scatter(accs[j], [dvec],
                                       jnp.maximum(accv, valj), mask=win)
                lose = jnp.logical_not(win)

                @pl.when(jnp.any(lose))
                def _():
                    def _cond(m):
                        return jnp.any(m)

                    def _body(m):
                        plsc.store_scatter(tmp, [dvec], lane, mask=m)
                        w2 = jnp.logical_and(
                            m, plsc.load_gather(tmp, [dvec]) == lane)
                        for j in range(4):
                            accv2 = plsc.load_gather(accs[j], [dvec])
                            plsc.store_scatter(accs[j], [dvec],
                                               jnp.maximum(accv2, vals[j]),
                                               mask=w2)
                        return jnp.logical_and(m, jnp.logical_not(w2))
                    lax.while_loop(_cond, _body, lose)
                return 0
            lax.fori_loop(0, NGROUP, _group, 0)

        _start(0, 0)
        _start(1, 1)

        def _pair(k, _):
            c0 = k * 2

            for b in range(2):
                _wait(b)
                _process(b)

                @pl.when(c0 + 2 + b < NCHUNK)
                def _():
                    _start(c0 + 2 + b, b)
            return 0
        lax.fori_loop(0, NCHUNK // 2, _pair, 0)

        for j in range(4):
            pltpu.sync_copy(accs[j], out_hbm.at[pl.ds((r0 + j) * N, N)])


def _sc_segmax(btall, edges, wall):
    mesh = plsc.VectorSubcoreMesh(core_axis_name="c", subcore_axis_name="s")
    return pl.kernel(
        _sc_body,
        out_type=jax.ShapeDtypeStruct((2 * H * N,), jnp.float32),
        mesh=mesh,
        compiler_params=pltpu.CompilerParams(needs_layout_passes=False),
        scratch_types=(
            [pltpu.VMEM((N,), jnp.float32) for _ in range(4)]
            + [pltpu.VMEM((N,), jnp.float32) for _ in range(4)]
            + [pltpu.VMEM((N,), jnp.int32),
               pltpu.VMEM((3 * CHUNK,), jnp.int32),
               pltpu.VMEM((3 * CHUNK,), jnp.int32),
               pltpu.VMEM((2 * H,), jnp.float32),
               pltpu.SemaphoreType.DMA,
               pltpu.SemaphoreType.DMA]
        ),
    )(btall, edges, wall)


def _tc_post_body(zb_ref, zf_ref, abt_b_ref, abt_f_ref, seg_ref,
                  u1_ref, u2_ref, ub_ref,
                  dzb_ref, dhb_ref, dbb_ref, dzf_ref, dhf_ref, dbf_ref,
                  twb_ref, tbb_ref, twf_ref, tbf_ref,
                  h_ref, y_ref, dist_ref, taub_ref, tauf_ref, hsb_ref, hsf_ref):
    i = pl.program_id(0)
    zb = zb_ref[...]
    zf = zf_ref[...]
    seg = seg_ref[...]
    aggrT_b = jnp.maximum(abt_b_ref[...] + seg[:H, :], 0.0)
    aggrT_f = jnp.maximum(abt_f_ref[...] + seg[H:, :], 0.0)
    u1 = u1_ref[...]
    u2 = u2_ref[...]
    ub = ub_ref[...]
    hb = jnp.maximum(jnp.dot(zb, u1, precision=_P)
                     + lax.dot_general(aggrT_b, u2, (((0,), (0,)), ((), ())),
                                       precision=_P) + ub, 0.0)
    hf = jnp.maximum(jnp.dot(zf, u1, precision=_P)
                     + lax.dot_general(aggrT_f, u2, (((0,), (0,)), ((), ())),
                                       precision=_P) + ub, 0.0)
    h_ref[...] = hb
    y_ref[...] = (jnp.dot(zb, dzb_ref[...], precision=_P)
                  + jnp.dot(hb, dhb_ref[...], precision=_P) + dbb_ref[...])
    dist_ref[...] = (jnp.dot(zf, dzf_ref[...], precision=_P)
                     + jnp.dot(hf, dhf_ref[...], precision=_P) + dbf_ref[...])

    valid = jnp.minimum(N - i * BN, BN)
    rmask = lax.broadcasted_iota(jnp.int32, (BN, 1), 0) < valid
    sb = jnp.sum(jnp.where(rmask, hb, 0.0), axis=0, keepdims=True)
    sf = jnp.sum(jnp.where(rmask, hf, 0.0), axis=0, keepdims=True)
    pad = jnp.zeros((7, H), jnp.float32)
    sb8 = jnp.concatenate([sb, pad], axis=0)
    sf8 = jnp.concatenate([sf, pad], axis=0)

    @pl.when(i == 0)
    def _():
        hsb_ref[...] = jnp.zeros((8, H), jnp.float32)
        hsf_ref[...] = jnp.zeros((8, H), jnp.float32)

    hsb_ref[...] += sb8
    hsf_ref[...] += sf8

    @pl.when(i == GRID - 1)
    def _():
        hmb = hsb_ref[0:1, :] * jnp.float32(1.0 / N)
        hmf = hsf_ref[0:1, :] * jnp.float32(1.0 / N)
        taub_ref[...] = jax.nn.sigmoid(
            jnp.dot(hmb, twb_ref[...], precision=_P) + tbb_ref[...])
        tauf_ref[...] = jax.nn.sigmoid(
            jnp.dot(hmf, twf_ref[...], precision=_P) + tbf_ref[...])


def _tc_post(zb, zf, abt_b, abt_f, seg, u1, u2, ub,
             dzb, dhb, dbb, dzf, dhf, dbf, twb, tbb, twf, tbf):
    row_specH = pl.BlockSpec((BN, H), lambda i: (i, 0))
    row_spec1 = pl.BlockSpec((BN, 1), lambda i: (i, 0))
    colT_spec = pl.BlockSpec((H, BN), lambda i: (0, i))
    colT2_spec = pl.BlockSpec((2 * H, BN), lambda i: (0, i))
    full = lambda shape: pl.BlockSpec(shape, lambda i: tuple(0 for _ in shape))
    return pl.pallas_call(
        _tc_post_body,
        grid=(GRID,),
        in_specs=[row_specH, row_specH, colT_spec, colT_spec, colT2_spec,
                  full((H, H)), full((H, H)), full((1, H)),
                  full((H, 1)), full((H, 1)), full((1, 1)),
                  full((H, 1)), full((H, 1)), full((1, 1)),
                  full((H, 1)), full((1, 1)), full((H, 1)), full((1, 1))],
        out_specs=[row_specH, row_spec1, row_spec1,
                   full((1, 1)), full((1, 1)), full((8, H)), full((8, H))],
        out_shape=[jax.ShapeDtypeStruct((N, H), jnp.float32),
                   jax.ShapeDtypeStruct((N, 1), jnp.float32),
                   jax.ShapeDtypeStruct((N, 1), jnp.float32),
                   jax.ShapeDtypeStruct((1, 1), jnp.float32),
                   jax.ShapeDtypeStruct((1, 1), jnp.float32),
                   jax.ShapeDtypeStruct((8, H), jnp.float32),
                   jax.ShapeDtypeStruct((8, H), jnp.float32)],
    )(zb, zf, abt_b, abt_f, seg, u1, u2, ub,
      dzb, dhb, dbb, dzf, dhf, dbf, twb, tbb, twf, tbf)


def kernel(bfs_x, bf_x, bfs_pre_h, bf_pre_h, edge_index, edge_attr, params):
    src = edge_index[0].astype(jnp.int32)
    dst = edge_index[1].astype(jnp.int32)
    attr = edge_attr.reshape(E).astype(jnp.float32)

    p = params
    ew0b = p['enc_bfs_W'][0:1]
    ewhb = p['enc_bfs_W'][1:]
    ebb = p['enc_bfs_b'].reshape(1, H)
    ew0f = p['enc_bf_W'][0:1]
    ewhf = p['enc_bf_W'][1:]
    ebf = p['enc_bf_b'].reshape(1, H)
    mw1 = p['M_W'][0:H]
    mw2 = p['M_W'][H:2 * H]
    wall = jnp.concatenate([p['M_W'][2 * H], p['M_W'][2 * H]], axis=0)
    mbt = p['M_b'].reshape(H, 1)
    u1 = p['U_W'][0:H]
    u2 = p['U_W'][H:]
    ub = p['U_b'].reshape(1, H)
    dzb = p['dec_bfs_W'][0:H]
    dhb = p['dec_bfs_W'][H:]
    dbb = p['dec_bfs_b'].reshape(1, 1)
    dzf = p['dec_bf_W'][0:H]
    dhf = p['dec_bf_W'][H:]
    dbf = p['dec_bf_b'].reshape(1, 1)
    twb = p['term_bfs_W'][0:H] + p['term_bfs_W'][H:]
    tbb = p['term_bfs_b'].reshape(1, 1)
    twf = p['term_bf_W'][0:H] + p['term_bf_W'][H:]
    tbf = p['term_bf_b'].reshape(1, 1)

    zb, zf, abt_b, abt_f, btall = _tc_pre(
        bfs_x, bf_x, bfs_pre_h, bf_pre_h,
        ew0b, ewhb, ebb, ew0f, ewhf, ebf, mw1, mw2, mbt)

    attr_bits = lax.bitcast_convert_type(attr, jnp.int32)
    edges = jnp.concatenate(
        [src.reshape(NCHUNK, CHUNK), dst.reshape(NCHUNK, CHUNK),
         attr_bits.reshape(NCHUNK, CHUNK)], axis=1).reshape(3 * E)
    seg = _sc_segmax(btall.reshape(2 * H * N), edges, wall)
    seg = seg.reshape(2 * H, N)

    bfs_h, y, dist, taub, tauf, _, _ = _tc_post(
        zb, zf, abt_b, abt_f, seg, u1, u2, ub,
        dzb, dhb, dbb, dzf, dhf, dbf, twb, tbb, twf, tbf)

    return (bfs_h, bfs_h, y, dist, taub, tauf)


# speedup vs baseline: 2.9373x; 1.0641x over previous
import functools

import jax
import jax.numpy as jnp
from jax import lax
from jax.experimental import pallas as pl
from jax.experimental.pallas import tpu as pltpu
from jax.experimental.pallas import tpu_sc as plsc

N = 10000
E = 320000
H = 128
BN = 1024
GRID = (N + BN - 1) // BN
CHUNK = 4000
NCHUNK = E // CHUNK
NGROUP = CHUNK // 16
NEG = float('-inf')
_P = jax.lax.Precision.HIGHEST


def _tc_pre_body(xb_ref, xf_ref, hb_ref, hf_ref,
                 ew0b_ref, ewhb_ref, ebb_ref, ew0f_ref, ewhf_ref, ebf_ref,
                 mw1_ref, mw2_ref, mbt_ref,
                 zb_ref, zf_ref, abt_b_ref, abt_f_ref, btall_ref):
    xb = xb_ref[...]
    xf = xf_ref[...]
    hb = hb_ref[...]
    hf = hf_ref[...]
    zb = jnp.maximum(xb * ew0b_ref[...] + jnp.dot(hb, ewhb_ref[...], precision=_P)
                     + ebb_ref[...], 0.0)
    zf = jnp.maximum(xf * ew0f_ref[...] + jnp.dot(hf, ewhf_ref[...], precision=_P)
                     + ebf_ref[...], 0.0)
    zb_ref[...] = zb
    zf_ref[...] = zf
    mw1 = mw1_ref[...]
    mw2 = mw2_ref[...]
    mbt = mbt_ref[...]
    abt_b_ref[...] = lax.dot_general(mw1, zb, (((0,), (1,)), ((), ())),
                                     precision=_P) + mbt
    abt_f_ref[...] = lax.dot_general(mw1, zf, (((0,), (1,)), ((), ())),
                                     precision=_P) + mbt
    bt_b = lax.dot_general(mw2, zb, (((0,), (1,)), ((), ())), precision=_P)
    bt_f = lax.dot_general(mw2, zf, (((0,), (1,)), ((), ())), precision=_P)
    btall_ref[...] = jnp.concatenate([bt_b, bt_f], axis=0)


def _tc_pre(bfs_x, bf_x, bfs_pre_h, bf_pre_h, ew0b, ewhb, ebb, ew0f, ewhf, ebf,
            mw1, mw2, mbt):
    row_spec1 = pl.BlockSpec((BN, 1), lambda i: (i, 0))
    row_specH = pl.BlockSpec((BN, H), lambda i: (i, 0))
    full = lambda shape: pl.BlockSpec(shape, lambda i: tuple(0 for _ in shape))
    colT_spec = pl.BlockSpec((H, BN), lambda i: (0, i))
    colT2_spec = pl.BlockSpec((2 * H, BN), lambda i: (0, i))
    return pl.pallas_call(
        _tc_pre_body,
        grid=(GRID,),
        in_specs=[row_spec1, row_spec1, row_specH, row_specH,
                  full((1, H)), full((H, H)), full((1, H)),
                  full((1, H)), full((H, H)), full((1, H)),
                  full((H, H)), full((H, H)), full((H, 1))],
        out_specs=[row_specH, row_specH, colT_spec, colT_spec, colT2_spec],
        out_shape=[jax.ShapeDtypeStruct((N, H), jnp.float32),
                   jax.ShapeDtypeStruct((N, H), jnp.float32),
                   jax.ShapeDtypeStruct((H, N), jnp.float32),
                   jax.ShapeDtypeStruct((H, N), jnp.float32),
                   jax.ShapeDtypeStruct((2 * H, N), jnp.float32)],
    )(bfs_x, bf_x, bfs_pre_h, bf_pre_h, ew0b, ewhb, ebb, ew0f, ewhf, ebf,
      mw1, mw2, mbt)


def _sc_body(bt_hbm, edges_hbm, w_hbm, out_hbm,
             bv0, bv1, bv2, bv3, acc0, acc1, acc2, acc3,
             tmp, tmp2, eb0, eb1, wv, sem0, sem1):
    wid = lax.axis_index("s") * 2 + lax.axis_index("c")
    pltpu.sync_copy(w_hbm, wv)
    lane = lax.iota(jnp.int32, 16)
    bvs = [bv0, bv1, bv2, bv3]
    accs = [acc0, acc1, acc2, acc3]
    ebufs = [eb0, eb1]
    sems = [sem0, sem1]
    W3 = 3 * CHUNK

    def _start(ci, b):
        pltpu.async_copy(edges_hbm.at[pl.ds(ci * W3, W3)], ebufs[b], sems[b])

    def _wait(b):
        pltpu.make_async_copy(edges_hbm.at[pl.ds(0, W3)], ebufs[b],
                              sems[b]).wait()

    for half in range(2):
        r0 = wid * 8 + half * 4
        for j in range(4):
            pltpu.sync_copy(bt_hbm.at[pl.ds((r0 + j) * N, N)], bvs[j])

        def _init(k, _):
            for j in range(4):
                accs[j][pl.ds(k * 16, 16)] = jnp.full((16,), NEG, jnp.float32)
            return 0
        lax.fori_loop(0, N // 16, _init, 0)

        wspl = [plsc.load_gather(wv, [jnp.broadcast_to(r0 + j, (16,)).astype(jnp.int32)])
                for j in range(4)]

        def _process(b):
            eb = ebufs[b]
            tmps = [tmp, tmp2]

            def _one(base, arb):
                svec = eb[pl.ds(base, 16)]
                dvec = eb[pl.ds(CHUNK + base, 16)]
                avec = plsc.bitcast(eb[pl.ds(2 * CHUNK + base, 16)],
                                    jnp.float32)
                plsc.store_scatter(arb, [dvec], lane)
                win = plsc.load_gather(arb, [dvec]) == lane
                vals = []
                for j in range(4):
                    bvj = plsc.load_gather(bvs[j], [svec])
                    valj = bvj + avec * wspl[j]
                    vals.append(valj)
                    accv = plsc.load_gather(accs[j], [dvec])
                    plsc.store_scatter(accs[j], [dvec],
                                       jnp.maximum(accv, valj), mask=win)
                lose = jnp.logical_not(win)

                @pl.when(jnp.any(lose))
                def _():
                    def _cond(m):
                        return jnp.any(m)

                    def _body(m):
                        plsc.store_scatter(arb, [dvec], lane, mask=m)
                        w2 = jnp.logical_and(
                            m, plsc.load_gather(arb, [dvec]) == lane)
                        for j in range(4):
                            accv2 = plsc.load_gather(accs[j], [dvec])
                            plsc.store_scatter(accs[j], [dvec],
                                               jnp.maximum(accv2, vals[j]),
                                               mask=w2)
                        return jnp.logical_and(m, jnp.logical_not(w2))
                    lax.while_loop(_cond, _body, lose)

            def _group2(u, _):
                for t in range(2):
                    _one((u * 2 + t) * 16, tmps[t])
                return 0
            lax.fori_loop(0, NGROUP // 2, _group2, 0)

        _start(0, 0)
        _start(1, 1)

        def _pair(k, _):
            c0 = k * 2

            for b in range(2):
                _wait(b)
                _process(b)

                @pl.when(c0 + 2 + b < NCHUNK)
                def _():
                    _start(c0 + 2 + b, b)
            return 0
        lax.fori_loop(0, NCHUNK // 2, _pair, 0)

        for j in range(4):
            pltpu.sync_copy(accs[j], out_hbm.at[pl.ds((r0 + j) * N, N)])


def _sc_segmax(btall, edges, wall):
    mesh = plsc.VectorSubcoreMesh(core_axis_name="c", subcore_axis_name="s")
    return pl.kernel(
        _sc_body,
        out_type=jax.ShapeDtypeStruct((2 * H * N,), jnp.float32),
        mesh=mesh,
        compiler_params=pltpu.CompilerParams(needs_layout_passes=False),
        scratch_types=(
            [pltpu.VMEM((N,), jnp.float32) for _ in range(4)]
            + [pltpu.VMEM((N,), jnp.float32) for _ in range(4)]
            + [pltpu.VMEM((N,), jnp.int32),
               pltpu.VMEM((N,), jnp.int32),
               pltpu.VMEM((3 * CHUNK,), jnp.int32),
               pltpu.VMEM((3 * CHUNK,), jnp.int32),
               pltpu.VMEM((2 * H,), jnp.float32),
               pltpu.SemaphoreType.DMA,
               pltpu.SemaphoreType.DMA]
        ),
    )(btall, edges, wall)


def _tc_post_body(zb_ref, zf_ref, abt_b_ref, abt_f_ref, seg_ref,
                  u1_ref, u2_ref, ub_ref,
                  dzb_ref, dhb_ref, dbb_ref, dzf_ref, dhf_ref, dbf_ref,
                  twb_ref, tbb_ref, twf_ref, tbf_ref,
                  h_ref, y_ref, dist_ref, taub_ref, tauf_ref, hsb_ref, hsf_ref):
    i = pl.program_id(0)
    zb = zb_ref[...]
    zf = zf_ref[...]
    seg = seg_ref[...]
    aggrT_b = jnp.maximum(abt_b_ref[...] + seg[:H, :], 0.0)
    aggrT_f = jnp.maximum(abt_f_ref[...] + seg[H:, :], 0.0)
    u1 = u1_ref[...]
    u2 = u2_ref[...]
    ub = ub_ref[...]
    hb = jnp.maximum(jnp.dot(zb, u1, precision=_P)
                     + lax.dot_general(aggrT_b, u2, (((0,), (0,)), ((), ())),
                                       precision=_P) + ub, 0.0)
    hf = jnp.maximum(jnp.dot(zf, u1, precision=_P)
                     + lax.dot_general(aggrT_f, u2, (((0,), (0,)), ((), ())),
                                       precision=_P) + ub, 0.0)
    h_ref[...] = hb
    y_ref[...] = (jnp.dot(zb, dzb_ref[...], precision=_P)
                  + jnp.dot(hb, dhb_ref[...], precision=_P) + dbb_ref[...])
    dist_ref[...] = (jnp.dot(zf, dzf_ref[...], precision=_P)
                     + jnp.dot(hf, dhf_ref[...], precision=_P) + dbf_ref[...])

    valid = jnp.minimum(N - i * BN, BN)
    rmask = lax.broadcasted_iota(jnp.int32, (BN, 1), 0) < valid
    sb = jnp.sum(jnp.where(rmask, hb, 0.0), axis=0, keepdims=True)
    sf = jnp.sum(jnp.where(rmask, hf, 0.0), axis=0, keepdims=True)
    pad = jnp.zeros((7, H), jnp.float32)
    sb8 = jnp.concatenate([sb, pad], axis=0)
    sf8 = jnp.concatenate([sf, pad], axis=0)

    @pl.when(i == 0)
    def _():
        hsb_ref[...] = jnp.zeros((8, H), jnp.float32)
        hsf_ref[...] = jnp.zeros((8, H), jnp.float32)

    hsb_ref[...] += sb8
    hsf_ref[...] += sf8

    @pl.when(i == GRID - 1)
    def _():
        hmb = hsb_ref[0:1, :] * jnp.float32(1.0 / N)
        hmf = hsf_ref[0:1, :] * jnp.float32(1.0 / N)
        taub_ref[...] = jax.nn.sigmoid(
            jnp.dot(hmb, twb_ref[...], precision=_P) + tbb_ref[...])
        tauf_ref[...] = jax.nn.sigmoid(
            jnp.dot(hmf, twf_ref[...], precision=_P) + tbf_ref[...])


def _tc_post(zb, zf, abt_b, abt_f, seg, u1, u2, ub,
             dzb, dhb, dbb, dzf, dhf, dbf, twb, tbb, twf, tbf):
    row_specH = pl.BlockSpec((BN, H), lambda i: (i, 0))
    row_spec1 = pl.BlockSpec((BN, 1), lambda i: (i, 0))
    colT_spec = pl.BlockSpec((H, BN), lambda i: (0, i))
    colT2_spec = pl.BlockSpec((2 * H, BN), lambda i: (0, i))
    full = lambda shape: pl.BlockSpec(shape, lambda i: tuple(0 for _ in shape))
    return pl.pallas_call(
        _tc_post_body,
        grid=(GRID,),
        in_specs=[row_specH, row_specH, colT_spec, colT_spec, colT2_spec,
                  full((H, H)), full((H, H)), full((1, H)),
                  full((H, 1)), full((H, 1)), full((1, 1)),
                  full((H, 1)), full((H, 1)), full((1, 1)),
                  full((H, 1)), full((1, 1)), full((H, 1)), full((1, 1))],
        out_specs=[row_specH, row_spec1, row_spec1,
                   full((1, 1)), full((1, 1)), full((8, H)), full((8, H))],
        out_shape=[jax.ShapeDtypeStruct((N, H), jnp.float32),
                   jax.ShapeDtypeStruct((N, 1), jnp.float32),
                   jax.ShapeDtypeStruct((N, 1), jnp.float32),
                   jax.ShapeDtypeStruct((1, 1), jnp.float32),
                   jax.ShapeDtypeStruct((1, 1), jnp.float32),
                   jax.ShapeDtypeStruct((8, H), jnp.float32),
                   jax.ShapeDtypeStruct((8, H), jnp.float32)],
    )(zb, zf, abt_b, abt_f, seg, u1, u2, ub,
      dzb, dhb, dbb, dzf, dhf, dbf, twb, tbb, twf, tbf)


def kernel(bfs_x, bf_x, bfs_pre_h, bf_pre_h, edge_index, edge_attr, params):
    src = edge_index[0].astype(jnp.int32)
    dst = edge_index[1].astype(jnp.int32)
    attr = edge_attr.reshape(E).astype(jnp.float32)

    p = params
    ew0b = p['enc_bfs_W'][0:1]
    ewhb = p['enc_bfs_W'][1:]
    ebb = p['enc_bfs_b'].reshape(1, H)
    ew0f = p['enc_bf_W'][0:1]
    ewhf = p['enc_bf_W'][1:]
    ebf = p['enc_bf_b'].reshape(1, H)
    mw1 = p['M_W'][0:H]
    mw2 = p['M_W'][H:2 * H]
    wall = jnp.concatenate([p['M_W'][2 * H], p['M_W'][2 * H]], axis=0)
    mbt = p['M_b'].reshape(H, 1)
    u1 = p['U_W'][0:H]
    u2 = p['U_W'][H:]
    ub = p['U_b'].reshape(1, H)
    dzb = p['dec_bfs_W'][0:H]
    dhb = p['dec_bfs_W'][H:]
    dbb = p['dec_bfs_b'].reshape(1, 1)
    dzf = p['dec_bf_W'][0:H]
    dhf = p['dec_bf_W'][H:]
    dbf = p['dec_bf_b'].reshape(1, 1)
    twb = p['term_bfs_W'][0:H] + p['term_bfs_W'][H:]
    tbb = p['term_bfs_b'].reshape(1, 1)
    twf = p['term_bf_W'][0:H] + p['term_bf_W'][H:]
    tbf = p['term_bf_b'].reshape(1, 1)

    zb, zf, abt_b, abt_f, btall = _tc_pre(
        bfs_x, bf_x, bfs_pre_h, bf_pre_h,
        ew0b, ewhb, ebb, ew0f, ewhf, ebf, mw1, mw2, mbt)

    attr_bits = lax.bitcast_convert_type(attr, jnp.int32)
    edges = jnp.concatenate(
        [src.reshape(NCHUNK, CHUNK), dst.reshape(NCHUNK, CHUNK),
         attr_bits.reshape(NCHUNK, CHUNK)], axis=1).reshape(3 * E)
    seg = _sc_segmax(btall.reshape(2 * H * N), edges, wall)
    seg = seg.reshape(2 * H, N)

    bfs_h, y, dist, taub, tauf, _, _ = _tc_post(
        zb, zf, abt_b, abt_f, seg, u1, u2, ub,
        dzb, dhb, dbb, dzf, dhf, dbf, twb, tbb, twf, tbf)

    return (bfs_h, bfs_h, y, dist, taub, tauf)


# speedup vs baseline: 4.2711x; 1.4541x over previous
import functools

import jax
import jax.numpy as jnp
from jax import lax
from jax.experimental import pallas as pl
from jax.experimental.pallas import tpu as pltpu
from jax.experimental.pallas import tpu_sc as plsc

N = 10000
E = 320000
H = 128
BN = 1024
GRID = (N + BN - 1) // BN
CHUNK = 4000
NCHUNK = E // CHUNK
NGROUP = CHUNK // 16
NEG = float('-inf')
_P = jax.lax.Precision.HIGHEST


def _tc_pre_body(xb_ref, xf_ref, hb_ref, hf_ref,
                 ew0b_ref, ewhb_ref, ebb_ref, ew0f_ref, ewhf_ref, ebf_ref,
                 mw1_ref, mw2_ref, mbt_ref,
                 zb_ref, zf_ref, abt_b_ref, abt_f_ref, btall_ref):
    xb = xb_ref[...]
    xf = xf_ref[...]
    hb = hb_ref[...]
    hf = hf_ref[...]
    zb = jnp.maximum(xb * ew0b_ref[...] + jnp.dot(hb, ewhb_ref[...], precision=_P)
                     + ebb_ref[...], 0.0)
    zf = jnp.maximum(xf * ew0f_ref[...] + jnp.dot(hf, ewhf_ref[...], precision=_P)
                     + ebf_ref[...], 0.0)
    zb_ref[...] = zb
    zf_ref[...] = zf
    mw1 = mw1_ref[...]
    mw2 = mw2_ref[...]
    mbt = mbt_ref[...]
    abt_b_ref[...] = lax.dot_general(mw1, zb, (((0,), (1,)), ((), ())),
                                     precision=_P) + mbt
    abt_f_ref[...] = lax.dot_general(mw1, zf, (((0,), (1,)), ((), ())),
                                     precision=_P) + mbt
    bt_b = lax.dot_general(mw2, zb, (((0,), (1,)), ((), ())), precision=_P)
    bt_f = lax.dot_general(mw2, zf, (((0,), (1,)), ((), ())), precision=_P)
    btall_ref[...] = jnp.concatenate([bt_b, bt_f], axis=0)


def _tc_pre(bfs_x, bf_x, bfs_pre_h, bf_pre_h, ew0b, ewhb, ebb, ew0f, ewhf, ebf,
            mw1, mw2, mbt):
    row_spec1 = pl.BlockSpec((BN, 1), lambda i: (i, 0))
    row_specH = pl.BlockSpec((BN, H), lambda i: (i, 0))
    full = lambda shape: pl.BlockSpec(shape, lambda i: tuple(0 for _ in shape))
    colT_spec = pl.BlockSpec((H, BN), lambda i: (0, i))
    colT2_spec = pl.BlockSpec((2 * H, BN), lambda i: (0, i))
    return pl.pallas_call(
        _tc_pre_body,
        grid=(GRID,),
        in_specs=[row_spec1, row_spec1, row_specH, row_specH,
                  full((1, H)), full((H, H)), full((1, H)),
                  full((1, H)), full((H, H)), full((1, H)),
                  full((H, H)), full((H, H)), full((H, 1))],
        out_specs=[row_specH, row_specH, colT_spec, colT_spec, colT2_spec],
        out_shape=[jax.ShapeDtypeStruct((N, H), jnp.float32),
                   jax.ShapeDtypeStruct((N, H), jnp.float32),
                   jax.ShapeDtypeStruct((H, N), jnp.float32),
                   jax.ShapeDtypeStruct((H, N), jnp.float32),
                   jax.ShapeDtypeStruct((2 * H, N), jnp.float32)],
    )(bfs_x, bf_x, bfs_pre_h, bf_pre_h, ew0b, ewhb, ebb, ew0f, ewhf, ebf,
      mw1, mw2, mbt)


NEGPAIR = -8454144


def _sc_body(bt_hbm, edges_hbm, w_hbm, out_hbm,
             bv0, bv1, bv2, bv3, acc0, acc1, acc2, acc3,
             tmp, tmp2, eb0, eb1, wv, sem0, sem1):
    wid = lax.axis_index("s") * 2 + lax.axis_index("c")
    pltpu.sync_copy(w_hbm, wv)
    lane = lax.iota(jnp.int32, 16)
    bvs = [bv0, bv1, bv2, bv3]
    accs = [acc0, acc1, acc2, acc3]
    ebufs = [eb0, eb1]
    sems = [sem0, sem1]
    W3 = 3 * CHUNK

    def _start(ci, b):
        pltpu.async_copy(edges_hbm.at[pl.ds(ci * W3, W3)], ebufs[b], sems[b])

    def _wait(b):
        pltpu.make_async_copy(edges_hbm.at[pl.ds(0, W3)], ebufs[b],
                              sems[b]).wait()

    r0 = wid * 4
    for j in range(4):
        pltpu.sync_copy(bt_hbm.at[pl.ds((r0 + j) * N, N)], bvs[j])

    def _init(k, _):
        for j in range(4):
            accs[j][pl.ds(k * 16, 16)] = jnp.full((16,), NEGPAIR, jnp.int32)
        return 0
    lax.fori_loop(0, N // 16, _init, 0)

    wspl = [(plsc.load_gather(
                wv, [jnp.broadcast_to(2 * (r0 + j), (16,)).astype(jnp.int32)]),
             plsc.load_gather(
                wv, [jnp.broadcast_to(2 * (r0 + j) + 1, (16,)).astype(jnp.int32)]))
            for j in range(4)]

    def _process(b):
        eb = ebufs[b]
        tmps = [tmp, tmp2]

        def _one(base, arb):
            svec = eb[pl.ds(base, 16)]
            dvec = eb[pl.ds(CHUNK + base, 16)]
            avec = plsc.bitcast(eb[pl.ds(2 * CHUNK + base, 16)],
                                jnp.float32)
            plsc.store_scatter(arb, [dvec], lane)
            win = plsc.load_gather(arb, [dvec]) == lane
            vals = []
            for j in range(4):
                blo, bhi = plsc.unpack(
                    plsc.bitcast(plsc.load_gather(bvs[j], [svec]),
                                 jnp.bfloat16),
                    format=plsc.PackFormat.INTERLEAVED)
                vlo = blo + avec * wspl[j][0]
                vhi = bhi + avec * wspl[j][1]
                vals.append((vlo, vhi))
                alo, ahi = plsc.unpack(
                    plsc.bitcast(plsc.load_gather(accs[j], [dvec]),
                                 jnp.bfloat16),
                    format=plsc.PackFormat.INTERLEAVED)
                nv = plsc.bitcast(
                    plsc.pack(jnp.maximum(alo, vlo), jnp.maximum(ahi, vhi),
                              format=plsc.PackFormat.INTERLEAVED), jnp.int32)
                plsc.store_scatter(accs[j], [dvec], nv, mask=win)
            lose = jnp.logical_not(win)

            @pl.when(jnp.any(lose))
            def _():
                def _cond(m):
                    return jnp.any(m)

                def _body(m):
                    plsc.store_scatter(arb, [dvec], lane, mask=m)
                    w2 = jnp.logical_and(
                        m, plsc.load_gather(arb, [dvec]) == lane)
                    for j in range(4):
                        alo2, ahi2 = plsc.unpack(
                            plsc.bitcast(plsc.load_gather(accs[j], [dvec]),
                                         jnp.bfloat16),
                            format=plsc.PackFormat.INTERLEAVED)
                        nv2 = plsc.bitcast(
                            plsc.pack(jnp.maximum(alo2, vals[j][0]),
                                      jnp.maximum(ahi2, vals[j][1]),
                                      format=plsc.PackFormat.INTERLEAVED),
                            jnp.int32)
                        plsc.store_scatter(accs[j], [dvec], nv2, mask=w2)
                    return jnp.logical_and(m, jnp.logical_not(w2))
                lax.while_loop(_cond, _body, lose)

        def _group2(u, _):
            for t in range(2):
                _one((u * 2 + t) * 16, tmps[t])
            return 0
        lax.fori_loop(0, NGROUP // 2, _group2, 0)

    _start(0, 0)
    _start(1, 1)

    def _pair(k, _):
        c0 = k * 2

        for b in range(2):
            _wait(b)
            _process(b)

            @pl.when(c0 + 2 + b < NCHUNK)
            def _():
                _start(c0 + 2 + b, b)
        return 0
    lax.fori_loop(0, NCHUNK // 2, _pair, 0)

    for j in range(4):
        pltpu.sync_copy(accs[j], out_hbm.at[pl.ds((r0 + j) * N, N)])


def _sc_segmax(btpacked, edges, wall):
    mesh = plsc.VectorSubcoreMesh(core_axis_name="c", subcore_axis_name="s")
    return pl.kernel(
        _sc_body,
        out_type=jax.ShapeDtypeStruct((H * N,), jnp.int32),
        mesh=mesh,
        compiler_params=pltpu.CompilerParams(needs_layout_passes=False),
        scratch_types=(
            [pltpu.VMEM((N,), jnp.int32) for _ in range(4)]
            + [pltpu.VMEM((N,), jnp.int32) for _ in range(4)]
            + [pltpu.VMEM((N,), jnp.int32),
               pltpu.VMEM((N,), jnp.int32),
               pltpu.VMEM((3 * CHUNK,), jnp.int32),
               pltpu.VMEM((3 * CHUNK,), jnp.int32),
               pltpu.VMEM((2 * H,), jnp.float32),
               pltpu.SemaphoreType.DMA,
               pltpu.SemaphoreType.DMA]
        ),
    )(btpacked, edges, wall)


def _tc_post_body(zb_ref, zf_ref, abt_b_ref, abt_f_ref, seg_ref,
                  u1_ref, u2_ref, ub_ref,
                  dzb_ref, dhb_ref, dbb_ref, dzf_ref, dhf_ref, dbf_ref,
                  twb_ref, tbb_ref, twf_ref, tbf_ref,
                  h_ref, y_ref, dist_ref, taub_ref, tauf_ref, hsb_ref, hsf_ref):
    i = pl.program_id(0)
    zb = zb_ref[...]
    zf = zf_ref[...]
    seg = seg_ref[...]
    aggrT_b = jnp.maximum(abt_b_ref[...] + seg[:H, :], 0.0)
    aggrT_f = jnp.maximum(abt_f_ref[...] + seg[H:, :], 0.0)
    u1 = u1_ref[...]
    u2 = u2_ref[...]
    ub = ub_ref[...]
    hb = jnp.maximum(jnp.dot(zb, u1, precision=_P)
                     + lax.dot_general(aggrT_b, u2, (((0,), (0,)), ((), ())),
                                       precision=_P) + ub, 0.0)
    hf = jnp.maximum(jnp.dot(zf, u1, precision=_P)
                     + lax.dot_general(aggrT_f, u2, (((0,), (0,)), ((), ())),
                                       precision=_P) + ub, 0.0)
    h_ref[...] = hb
    y_ref[...] = (jnp.dot(zb, dzb_ref[...], precision=_P)
                  + jnp.dot(hb, dhb_ref[...], precision=_P) + dbb_ref[...])
    dist_ref[...] = (jnp.dot(zf, dzf_ref[...], precision=_P)
                     + jnp.dot(hf, dhf_ref[...], precision=_P) + dbf_ref[...])

    valid = jnp.minimum(N - i * BN, BN)
    rmask = lax.broadcasted_iota(jnp.int32, (BN, 1), 0) < valid
    sb = jnp.sum(jnp.where(rmask, hb, 0.0), axis=0, keepdims=True)
    sf = jnp.sum(jnp.where(rmask, hf, 0.0), axis=0, keepdims=True)
    pad = jnp.zeros((7, H), jnp.float32)
    sb8 = jnp.concatenate([sb, pad], axis=0)
    sf8 = jnp.concatenate([sf, pad], axis=0)

    @pl.when(i == 0)
    def _():
        hsb_ref[...] = jnp.zeros((8, H), jnp.float32)
        hsf_ref[...] = jnp.zeros((8, H), jnp.float32)

    hsb_ref[...] += sb8
    hsf_ref[...] += sf8

    @pl.when(i == GRID - 1)
    def _():
        hmb = hsb_ref[0:1, :] * jnp.float32(1.0 / N)
        hmf = hsf_ref[0:1, :] * jnp.float32(1.0 / N)
        taub_ref[...] = jax.nn.sigmoid(
            jnp.dot(hmb, twb_ref[...], precision=_P) + tbb_ref[...])
        tauf_ref[...] = jax.nn.sigmoid(
            jnp.dot(hmf, twf_ref[...], precision=_P) + tbf_ref[...])


def _tc_post(zb, zf, abt_b, abt_f, seg, u1, u2, ub,
             dzb, dhb, dbb, dzf, dhf, dbf, twb, tbb, twf, tbf):
    row_specH = pl.BlockSpec((BN, H), lambda i: (i, 0))
    row_spec1 = pl.BlockSpec((BN, 1), lambda i: (i, 0))
    colT_spec = pl.BlockSpec((H, BN), lambda i: (0, i))
    colT2_spec = pl.BlockSpec((2 * H, BN), lambda i: (0, i))
    full = lambda shape: pl.BlockSpec(shape, lambda i: tuple(0 for _ in shape))
    return pl.pallas_call(
        _tc_post_body,
        grid=(GRID,),
        in_specs=[row_specH, row_specH, colT_spec, colT_spec, colT2_spec,
                  full((H, H)), full((H, H)), full((1, H)),
                  full((H, 1)), full((H, 1)), full((1, 1)),
                  full((H, 1)), full((H, 1)), full((1, 1)),
                  full((H, 1)), full((1, 1)), full((H, 1)), full((1, 1))],
        out_specs=[row_specH, row_spec1, row_spec1,
                   full((1, 1)), full((1, 1)), full((8, H)), full((8, H))],
        out_shape=[jax.ShapeDtypeStruct((N, H), jnp.float32),
                   jax.ShapeDtypeStruct((N, 1), jnp.float32),
                   jax.ShapeDtypeStruct((N, 1), jnp.float32),
                   jax.ShapeDtypeStruct((1, 1), jnp.float32),
                   jax.ShapeDtypeStruct((1, 1), jnp.float32),
                   jax.ShapeDtypeStruct((8, H), jnp.float32),
                   jax.ShapeDtypeStruct((8, H), jnp.float32)],
    )(zb, zf, abt_b, abt_f, seg, u1, u2, ub,
      dzb, dhb, dbb, dzf, dhf, dbf, twb, tbb, twf, tbf)


def kernel(bfs_x, bf_x, bfs_pre_h, bf_pre_h, edge_index, edge_attr, params):
    src = edge_index[0].astype(jnp.int32)
    dst = edge_index[1].astype(jnp.int32)
    attr = edge_attr.reshape(E).astype(jnp.float32)

    p = params
    ew0b = p['enc_bfs_W'][0:1]
    ewhb = p['enc_bfs_W'][1:]
    ebb = p['enc_bfs_b'].reshape(1, H)
    ew0f = p['enc_bf_W'][0:1]
    ewhf = p['enc_bf_W'][1:]
    ebf = p['enc_bf_b'].reshape(1, H)
    mw1 = p['M_W'][0:H]
    mw2 = p['M_W'][H:2 * H]
    wall = jnp.concatenate([p['M_W'][2 * H], p['M_W'][2 * H]], axis=0)
    mbt = p['M_b'].reshape(H, 1)
    u1 = p['U_W'][0:H]
    u2 = p['U_W'][H:]
    ub = p['U_b'].reshape(1, H)
    dzb = p['dec_bfs_W'][0:H]
    dhb = p['dec_bfs_W'][H:]
    dbb = p['dec_bfs_b'].reshape(1, 1)
    dzf = p['dec_bf_W'][0:H]
    dhf = p['dec_bf_W'][H:]
    dbf = p['dec_bf_b'].reshape(1, 1)
    twb = p['term_bfs_W'][0:H] + p['term_bfs_W'][H:]
    tbb = p['term_bfs_b'].reshape(1, 1)
    twf = p['term_bf_W'][0:H] + p['term_bf_W'][H:]
    tbf = p['term_bf_b'].reshape(1, 1)

    zb, zf, abt_b, abt_f, btall = _tc_pre(
        bfs_x, bf_x, bfs_pre_h, bf_pre_h,
        ew0b, ewhb, ebb, ew0f, ewhf, ebf, mw1, mw2, mbt)

    attr_bits = lax.bitcast_convert_type(attr, jnp.int32)
    edges = jnp.concatenate(
        [src.reshape(NCHUNK, CHUNK), dst.reshape(NCHUNK, CHUNK),
         attr_bits.reshape(NCHUNK, CHUNK)], axis=1).reshape(3 * E)
    vb = jnp.swapaxes(btall.reshape(H, 2, N).astype(jnp.bfloat16), 1, 2)
    btp = lax.bitcast_convert_type(vb, jnp.int32).reshape(H * N)
    segp = _sc_segmax(btp, edges, wall).reshape(H, N)
    seg = jnp.swapaxes(
        lax.bitcast_convert_type(segp, jnp.bfloat16), 1, 2
    ).reshape(2 * H, N).astype(jnp.float32)

    bfs_h, y, dist, taub, tauf, _, _ = _tc_post(
        zb, zf, abt_b, abt_f, seg, u1, u2, ub,
        dzb, dhb, dbb, dzf, dhf, dbf, twb, tbb, twf, tbf)

    return (bfs_h, bfs_h, y, dist, taub, tauf)
